# Initial kernel scaffold; baseline (speedup 1.0000x reference)
import functools

import jax
import jax.numpy as jnp
from jax import lax
from jax.experimental import pallas as pl
from jax.experimental.pallas import tpu as pltpu
from jax.experimental.pallas import tpu_sc as plsc

RES = 128
D = 1024
B = 262144
L = 16
NC, NS = 2, 16
NW = NC * NS
PTS = B // NW
CH = 16
NCHUNK = PTS // CH

_mesh = plsc.VectorSubcoreMesh(core_axis_name="c", subcore_axis_name="s")


@functools.partial(
    pl.kernel,
    out_type=jax.ShapeDtypeStruct((B, D), jnp.float32),
    mesh=_mesh,
    scratch_types=[
        pltpu.VMEM((PTS,), jnp.float32),
        pltpu.VMEM((PTS,), jnp.float32),
        pltpu.VMEM((4 * CH,), jnp.int32),
        pltpu.VMEM((4 * CH,), jnp.float32),
        pltpu.VMEM((4 * CH, D), jnp.float32),
        pltpu.VMEM((CH, D), jnp.float32),
        pltpu.SemaphoreType.DMA,
    ],
)
def _dense_map_sc(xs_hbm, ys_hbm, table_hbm, out_hbm,
                  xs_v, ys_v, idx_v, w_v, rows_v, out_v, sem):
    wid = lax.axis_index("s") * NC + lax.axis_index("c")
    base = wid * PTS
    pltpu.sync_copy(xs_hbm.at[pl.ds(base, PTS)], xs_v)
    pltpu.sync_copy(ys_hbm.at[pl.ds(base, PTS)], ys_v)

    def chunk_body(c, _):
        off = c * CH
        x = xs_v[pl.ds(off, L)] * (RES - 1.0)
        y = ys_v[pl.ds(off, L)] * (RES - 1.0)
        xi = x.astype(jnp.int32)
        yi = y.astype(jnp.int32)
        xf = x - xi.astype(jnp.float32)
        yf = y - yi.astype(jnp.float32)
        cell = xi * RES + yi
        idx_v[pl.ds(0, L)] = cell
        idx_v[pl.ds(16, L)] = cell + RES
        idx_v[pl.ds(32, L)] = cell + 1
        idx_v[pl.ds(48, L)] = cell + RES + 1
        gx = 1.0 - xf
        gy = 1.0 - yf
        w_v[pl.ds(0, L)] = gx * gy
        w_v[pl.ds(16, L)] = xf * gy
        w_v[pl.ds(32, L)] = gx * yf
        w_v[pl.ds(48, L)] = xf * yf
        pltpu.async_copy(table_hbm.at[idx_v], rows_v, sem).wait()

        def p_body(p, _):
            pv = jnp.full((L,), p, jnp.int32)
            w0 = plsc.load_gather(w_v, [pv])
            w1 = plsc.load_gather(w_v, [pv + 16])
            w2 = plsc.load_gather(w_v, [pv + 32])
            w3 = plsc.load_gather(w_v, [pv + 48])

            def j_body(j, _):
                col = j * L
                r0 = rows_v[p, pl.ds(col, L)]
                r1 = rows_v[p + 16, pl.ds(col, L)]
                r2 = rows_v[p + 32, pl.ds(col, L)]
                r3 = rows_v[p + 48, pl.ds(col, L)]
                out_v[p, pl.ds(col, L)] = (w0 * r0 + w1 * r1) + (w2 * r2 + w3 * r3)
                return 0

            lax.fori_loop(0, D // L, j_body, 0, unroll=4)
            return 0

        lax.fori_loop(0, CH, p_body, 0)
        pltpu.sync_copy(out_v, out_hbm.at[pl.ds(base + off, CH)])
        return 0

    lax.fori_loop(0, NCHUNK, chunk_body, 0)


def kernel(inputs, embeddings):
    xs = inputs[:, 0]
    ys = inputs[:, 1]
    return _dense_map_sc(xs, ys, embeddings)


# speedup vs baseline: 1.6763x; 1.6763x over previous
import functools

import jax
import jax.numpy as jnp
from jax import lax
from jax.experimental import pallas as pl
from jax.experimental.pallas import tpu as pltpu
from jax.experimental.pallas import tpu_sc as plsc

RES = 128
D = 1024
B = 262144
L = 16
NC, NS = 2, 16
NW = NC * NS
PTS = B // NW
CH = 16
NCHUNK = PTS // CH

_mesh = plsc.VectorSubcoreMesh(core_axis_name="c", subcore_axis_name="s")


@functools.partial(
    pl.kernel,
    out_type=jax.ShapeDtypeStruct((B, D), jnp.float32),
    mesh=_mesh,
    scratch_types=[
        pltpu.VMEM((PTS,), jnp.float32),
        pltpu.VMEM((PTS,), jnp.float32),
        pltpu.VMEM((4 * CH,), jnp.int32),
        pltpu.VMEM((4 * CH, D), jnp.float32),
        pltpu.VMEM((CH, D), jnp.float32),
        pltpu.SemaphoreType.DMA,
    ],
)
def _dense_map_sc(xs_hbm, ys_hbm, table_hbm, out_hbm,
                  xs_v, ys_v, idx_v, rows_v, out_v, sem):
    wid = lax.axis_index("s") * NC + lax.axis_index("c")
    base = wid * PTS
    pltpu.sync_copy(xs_hbm.at[pl.ds(base, PTS)], xs_v)
    pltpu.sync_copy(ys_hbm.at[pl.ds(base, PTS)], ys_v)

    def chunk_body(c, _):
        off = c * CH
        x = xs_v[pl.ds(off, L)] * (RES - 1.0)
        y = ys_v[pl.ds(off, L)] * (RES - 1.0)
        xi = x.astype(jnp.int32)
        yi = y.astype(jnp.int32)
        xf = x - xi.astype(jnp.float32)
        yf = y - yi.astype(jnp.float32)
        cell = xi * RES + yi
        idx_v[pl.ds(0, L)] = cell
        idx_v[pl.ds(16, L)] = cell + RES
        idx_v[pl.ds(32, L)] = cell + 1
        idx_v[pl.ds(48, L)] = cell + RES + 1
        gx = 1.0 - xf
        gy = 1.0 - yf
        wa = gx * gy
        wb = xf * gy
        wc = gx * yf
        wd = xf * yf
        pltpu.async_copy(table_hbm.at[idx_v], rows_v, sem).wait()

        for p in range(CH):
            w0 = jnp.full((L,), wa[p])
            w1 = jnp.full((L,), wb[p])
            w2 = jnp.full((L,), wc[p])
            w3 = jnp.full((L,), wd[p])

            def j_body(j, _):
                col = j * L
                r0 = rows_v[p, pl.ds(col, L)]
                r1 = rows_v[p + 16, pl.ds(col, L)]
                r2 = rows_v[p + 32, pl.ds(col, L)]
                r3 = rows_v[p + 48, pl.ds(col, L)]
                out_v[p, pl.ds(col, L)] = (w0 * r0 + w1 * r1) + (w2 * r2 + w3 * r3)
                return 0

            lax.fori_loop(0, D // L, j_body, 0, unroll=4)
        pltpu.sync_copy(out_v, out_hbm.at[pl.ds(base + off, CH)])
        return 0

    lax.fori_loop(0, NCHUNK, chunk_body, 0)


def kernel(inputs, embeddings):
    xs = inputs[:, 0]
    ys = inputs[:, 1]
    return _dense_map_sc(xs, ys, embeddings)
